# Initial kernel scaffold; baseline (speedup 1.0000x reference)
import jax
import jax.numpy as jnp
from jax.experimental import pallas as pl


def kernel(x, edge_index, edge_attr, batch, nt_W1, nt_b1, nt_W2, nt_b2, et_W1, et_b1, et_W2, et_b2, conv_lin_W, conv_lin_b, conv_eps, conv_W1, conv_b1, conv_W2, conv_b2, h_W1, h_b1, h_W2, h_b2):
    raise NotImplementedError("write your pallas kernel here")


# speedup vs baseline: 2.3538x; 2.3538x over previous
import functools

import jax
import jax.numpy as jnp
from jax import lax
from jax.experimental import pallas as pl
from jax.experimental.pallas import tpu as pltpu
from jax.experimental.pallas import tpu_sc as plsc

N = 10000
E = 320000
D = 128
DE = 16
G = 64

NC, NS = 2, 16
NW = NC * NS
CHUNK = 128
CPW = 80
EP = NW * CPW * CHUNK
NPAD = 10112
RPT = NPAD // NS
BE = 1280

_SQRT_HALF = 0.7071067811865476


def _gelu(v):
    return 0.5 * v * (1.0 + lax.erf(v * _SQRT_HALF))


def _nt_body(x_ref, w1_ref, b1_ref, w2_ref, b2_ref, o_ref):
    h = _gelu(jnp.dot(x_ref[...], w1_ref[...],
                      preferred_element_type=jnp.float32) + b1_ref[...])
    o_ref[...] = _gelu(jnp.dot(h, w2_ref[...],
                               preferred_element_type=jnp.float32) + b2_ref[...])


def _et_body(attr_ref, ew1_ref, eb1_ref, ew2_ref, eb2_ref, lw_ref, lb_ref,
             e0_ref, e1_ref, e2_ref):
    ea = _gelu(jnp.dot(attr_ref[...], ew1_ref[...],
                       preferred_element_type=jnp.float32) + eb1_ref[...])
    ea = _gelu(jnp.dot(ea, ew2_ref[...],
                       preferred_element_type=jnp.float32) + eb2_ref[...])
    for l, o_ref in enumerate((e0_ref, e1_ref, e2_ref)):
        o_ref[...] = jnp.dot(ea, lw_ref[l],
                             preferred_element_type=jnp.float32) + lb_ref[l]


def _upd_body(h_ref, a_ref, w1_ref, b1_ref, w2_ref, b2_ref, eps_ref, o_ref,
              *, final_relu):
    z = eps_ref[0, 0] * h_ref[...] + a_ref[0, :N, :] + a_ref[1, :N, :]
    z = jnp.maximum(jnp.dot(z, w1_ref[...],
                            preferred_element_type=jnp.float32) + b1_ref[...], 0.0)
    z = jnp.dot(z, w2_ref[...], preferred_element_type=jnp.float32) + b2_ref[...]
    o_ref[...] = jnp.maximum(z, 0.0) if final_relu else z


def _final_body(h_ref, a_ref, w1_ref, b1_ref, w2_ref, b2_ref, eps_ref,
                batch_ref, hw1_ref, hb1_ref, hw2_ref, hb2_ref, o_ref):
    z = eps_ref[0, 0] * h_ref[...] + a_ref[0, :N, :] + a_ref[1, :N, :]
    z = jnp.maximum(jnp.dot(z, w1_ref[...],
                            preferred_element_type=jnp.float32) + b1_ref[...], 0.0)
    z = jnp.dot(z, w2_ref[...], preferred_element_type=jnp.float32) + b2_ref[...]
    gid = lax.broadcasted_iota(jnp.int32, (G, N), 0)
    oh = jnp.where(gid == batch_ref[...], 1.0, 0.0)
    sums = jnp.dot(oh, z, preferred_element_type=jnp.float32)
    cnt = jnp.sum(oh, axis=1, keepdims=True)
    pooled = sums / jnp.maximum(cnt, 1.0)
    p = jnp.maximum(jnp.dot(pooled, hw1_ref[...],
                            preferred_element_type=jnp.float32) + hb1_ref[...], 0.0)
    o_ref[...] = jnp.dot(p, hw2_ref[...],
                         preferred_element_type=jnp.float32) + hb2_ref[...]


def _sc_msg_body(e_hbm, h_hbm, src_hbm, dst_hbm, out_hbm, src_v, dst_v, buf, acc, sem):
    c = lax.axis_index("c")
    s = lax.axis_index("s")
    wid = s * NC + c

    zero = jnp.zeros((16,), jnp.float32)

    def _zero_row(r, carry):
        for j in range(D // 16):
            buf[r, pl.ds(j * 16, 16)] = zero
        return carry

    lax.fori_loop(0, CHUNK, _zero_row, 0)
    base = s * RPT
    off = 0
    for sz in (128, 128, 128, 128, RPT - 4 * 128):
        pltpu.sync_copy(buf.at[pl.ds(0, sz)], acc.at[pl.ds(base + off, sz)])
        off += sz
    plsc.subcore_barrier()

    pltpu.sync_copy(src_hbm.at[pl.ds(wid * CPW, CPW)], src_v)
    pltpu.sync_copy(dst_hbm.at[pl.ds(wid * CPW, CPW)], dst_v)

    def _chunk(k, carry):
        ebase = pl.multiple_of((wid * CPW + k) * CHUNK, CHUNK)
        pltpu.sync_copy(e_hbm.at[pl.ds(ebase, CHUNK)], buf)
        pltpu.async_copy(h_hbm.at[src_v.at[k]], buf, sem, add=True).wait()

        def _relu_row(r, inner):
            for j in range(D // 16):
                sl = pl.ds(j * 16, 16)
                buf[r, sl] = jnp.maximum(buf[r, sl], 0.0)
            return inner

        lax.fori_loop(0, CHUNK, _relu_row, 0)
        pltpu.sync_copy(buf, acc.at[dst_v.at[k]], add=True)
        return carry

    lax.fori_loop(0, CPW, _chunk, 0)
    plsc.subcore_barrier()

    pltpu.sync_copy(acc.at[pl.ds(s * RPT, RPT)],
                    out_hbm.at[c, pl.ds(s * RPT, RPT)])


@functools.cache
def _sc_msg_kernel():
    mesh = plsc.VectorSubcoreMesh(core_axis_name="c", subcore_axis_name="s")
    return pl.kernel(
        _sc_msg_body,
        out_type=jax.ShapeDtypeStruct((NC, NPAD, D), jnp.float32),
        mesh=mesh,
        scratch_types=[
            pltpu.VMEM((CPW, CHUNK), jnp.int32),
            pltpu.VMEM((CPW, CHUNK), jnp.int32),
            pltpu.VMEM((CHUNK, D), jnp.float32),
            pltpu.VMEM_SHARED((NPAD, D), jnp.float32),
            pltpu.SemaphoreType.DMA,
        ],
    )


def _sc_msg(e, h, src2, dst2):
    return _sc_msg_kernel()(e, h, src2, dst2)


def kernel(x, edge_index, edge_attr, batch, nt_W1, nt_b1, nt_W2, nt_b2,
           et_W1, et_b1, et_W2, et_b2, conv_lin_W, conv_lin_b, conv_eps,
           conv_W1, conv_b1, conv_W2, conv_b2, h_W1, h_b1, h_W2, h_b2):
    f32 = jnp.float32
    src2 = jnp.pad(edge_index[0], (0, EP - E)).reshape(NW * CPW, CHUNK)
    dst2 = jnp.pad(edge_index[1], (0, EP - E),
                   constant_values=N).reshape(NW * CPW, CHUNK)
    attr = jnp.pad(edge_attr, ((0, EP - E), (0, 0)))

    h0 = pl.pallas_call(
        _nt_body,
        out_shape=jax.ShapeDtypeStruct((N, D), f32),
    )(x, nt_W1, nt_b1.reshape(1, D), nt_W2, nt_b2.reshape(1, D))

    full2 = lambda i: (0, 0)
    full3 = lambda i: (0, 0, 0)
    e_all = pl.pallas_call(
        _et_body,
        grid=(EP // BE,),
        in_specs=[
            pl.BlockSpec((BE, DE), lambda i: (i, 0)),
            pl.BlockSpec((DE, D), full2),
            pl.BlockSpec((1, D), full2),
            pl.BlockSpec((D, D), full2),
            pl.BlockSpec((1, D), full2),
            pl.BlockSpec((3, D, D), full3),
            pl.BlockSpec((3, 1, D), full3),
        ],
        out_specs=[pl.BlockSpec((BE, D), lambda i: (i, 0))] * 3,
        out_shape=[jax.ShapeDtypeStruct((EP, D), f32)] * 3,
    )(attr, et_W1, et_b1.reshape(1, D), et_W2, et_b2.reshape(1, D),
      conv_lin_W, conv_lin_b.reshape(3, 1, D))

    h = h0
    for l in range(2):
        agg = _sc_msg(e_all[l], h, src2, dst2)
        h = pl.pallas_call(
            functools.partial(_upd_body, final_relu=l < 2),
            out_shape=jax.ShapeDtypeStruct((N, D), f32),
        )(h, agg, conv_W1[l], conv_b1[l].reshape(1, D), conv_W2[l],
          conv_b2[l].reshape(1, D),
          (1.0 + conv_eps[l]).reshape(1, 1))

    agg = _sc_msg(e_all[2], h, src2, dst2)
    out = pl.pallas_call(
        _final_body,
        out_shape=jax.ShapeDtypeStruct((G, 1), f32),
    )(h, agg, conv_W1[2], conv_b1[2].reshape(1, D), conv_W2[2],
      conv_b2[2].reshape(1, D), (1.0 + conv_eps[2]).reshape(1, 1),
      batch.reshape(1, N), h_W1, h_b1.reshape(1, D), h_W2, h_b2.reshape(1, 1))
    return out


# speedup vs baseline: 2.7495x; 1.1681x over previous
import functools

import jax
import jax.numpy as jnp
from jax import lax
from jax.experimental import pallas as pl
from jax.experimental.pallas import tpu as pltpu
from jax.experimental.pallas import tpu_sc as plsc

N = 10000
E = 320000
D = 128
DE = 16
G = 64

NC, NS = 2, 16
NW = NC * NS
CHUNK = 128
CPW = 80
EP = NW * CPW * CHUNK
NPAD = 10112
RPT = NPAD // NS
BE = 1280

_SQRT_HALF = 0.7071067811865476


def _gelu(v):
    return 0.5 * v * (1.0 + lax.erf(v * _SQRT_HALF))


def _nt_body(x_ref, w1_ref, b1_ref, w2_ref, b2_ref, o_ref):
    h = _gelu(jnp.dot(x_ref[...], w1_ref[...],
                      preferred_element_type=jnp.float32) + b1_ref[...])
    o_ref[...] = _gelu(jnp.dot(h, w2_ref[...],
                               preferred_element_type=jnp.float32) + b2_ref[...])


def _et_body(attr_ref, ew1_ref, eb1_ref, ew2_ref, eb2_ref, lw_ref, lb_ref,
             e0_ref, e1_ref, e2_ref):
    ea = _gelu(jnp.dot(attr_ref[...], ew1_ref[...],
                       preferred_element_type=jnp.float32) + eb1_ref[...])
    ea = _gelu(jnp.dot(ea, ew2_ref[...],
                       preferred_element_type=jnp.float32) + eb2_ref[...])
    for l, o_ref in enumerate((e0_ref, e1_ref, e2_ref)):
        o_ref[...] = jnp.dot(ea, lw_ref[l],
                             preferred_element_type=jnp.float32) + lb_ref[l]


def _upd_body(h_ref, a_ref, w1_ref, b1_ref, w2_ref, b2_ref, eps_ref, o_ref,
              *, final_relu):
    z = eps_ref[0, 0] * h_ref[...] + a_ref[0, :N, :] + a_ref[1, :N, :]
    z = jnp.maximum(jnp.dot(z, w1_ref[...],
                            preferred_element_type=jnp.float32) + b1_ref[...], 0.0)
    z = jnp.dot(z, w2_ref[...], preferred_element_type=jnp.float32) + b2_ref[...]
    o_ref[...] = jnp.maximum(z, 0.0) if final_relu else z


def _final_body(h_ref, a_ref, w1_ref, b1_ref, w2_ref, b2_ref, eps_ref,
                batch_ref, hw1_ref, hb1_ref, hw2_ref, hb2_ref, o_ref):
    z = eps_ref[0, 0] * h_ref[...] + a_ref[0, :N, :] + a_ref[1, :N, :]
    z = jnp.maximum(jnp.dot(z, w1_ref[...],
                            preferred_element_type=jnp.float32) + b1_ref[...], 0.0)
    z = jnp.dot(z, w2_ref[...], preferred_element_type=jnp.float32) + b2_ref[...]
    gid = lax.broadcasted_iota(jnp.int32, (G, N), 0)
    oh = jnp.where(gid == batch_ref[...], 1.0, 0.0)
    sums = jnp.dot(oh, z, preferred_element_type=jnp.float32)
    cnt = jnp.sum(oh, axis=1, keepdims=True)
    pooled = sums / jnp.maximum(cnt, 1.0)
    p = jnp.maximum(jnp.dot(pooled, hw1_ref[...],
                            preferred_element_type=jnp.float32) + hb1_ref[...], 0.0)
    o_ref[...] = jnp.dot(p, hw2_ref[...],
                         preferred_element_type=jnp.float32) + hb2_ref[...]


NBUF = 3
ZR = RPT // 8


def _sc_msg_body(e_hbm, h_hbm, idx_hbm, out_hbm, idx_v, bufs, acc,
                 sem_e, sem_g, sem_s, sem_i):
    c = lax.axis_index("c")
    s = lax.axis_index("s")
    wid = s * NC + c
    cbase = wid * CPW

    def issue_eload(k, j):
        ebase = pl.multiple_of((cbase + k) * CHUNK, CHUNK)
        pltpu.async_copy(e_hbm.at[pl.ds(ebase, CHUNK)], bufs.at[j], sem_e)

    def wait_eload():
        pltpu.make_async_copy(e_hbm.at[pl.ds(0, CHUNK)], bufs.at[0],
                              sem_e).wait()

    def issue_idx(k, j):
        pltpu.async_copy(idx_hbm.at[wid, :, k], idx_v.at[j], sem_i)

    def wait_idx():
        pltpu.make_async_copy(idx_hbm.at[0, :, 0], idx_v.at[0], sem_i).wait()

    def issue_gather(j):
        pltpu.async_copy(h_hbm.at[idx_v.at[j, 0]], bufs.at[j], sem_g,
                         add=True)

    def wait_gather():
        pltpu.make_async_copy(h_hbm.at[idx_v.at[0, 0]], bufs.at[0],
                              sem_g).wait()

    def issue_scatter(j):
        pltpu.async_copy(bufs.at[j], acc.at[idx_v.at[j, 1]], sem_s, add=True)

    def wait_scatter():
        pltpu.make_async_copy(bufs.at[0], acc.at[idx_v.at[0, 1]],
                              sem_s).wait()

    def relu(j):
        def _relu_row(r, inner):
            for q in range(D // 16):
                sl = pl.ds(q * 16, 16)
                bufs[j, r, sl] = jnp.maximum(bufs[j, r, sl], 0.0)
            return inner

        lax.fori_loop(0, CHUNK, _relu_row, 0, unroll=4)

    zero = jnp.zeros((16,), jnp.float32)

    def _zero_row(r, carry):
        for q in range(D // 16):
            bufs[0, r, pl.ds(q * 16, 16)] = zero
        return carry

    lax.fori_loop(0, ZR, _zero_row, 0)
    base = s * RPT

    def _zfill(i, carry):
        pltpu.sync_copy(bufs.at[0, pl.ds(0, ZR)],
                        acc.at[pl.ds(base + i * ZR, ZR)])
        return carry

    lax.fori_loop(0, 8, _zfill, 0)
    plsc.subcore_barrier()

    def _prime(k, carry):
        issue_idx(k, k)
        issue_eload(k, k)
        return carry

    lax.fori_loop(0, NBUF, _prime, 0)
    wait_eload()
    wait_idx()
    issue_gather(0)

    def _step(k, carry):
        j = lax.rem(k, NBUF)
        jn = lax.rem(k + 1, NBUF)

        @pl.when(k + 1 < CPW)
        def _():
            wait_eload()
            wait_idx()
            issue_gather(jn)

        wait_gather()
        relu(j)
        issue_scatter(j)

        @pl.when(k >= 1)
        def _():
            wait_scatter()

            @pl.when(k - 1 + NBUF < CPW)
            def _():
                jp = lax.rem(k - 1, NBUF)
                issue_idx(k - 1 + NBUF, jp)
                issue_eload(k - 1 + NBUF, jp)

        return carry

    lax.fori_loop(0, CPW, _step, 0)
    wait_scatter()
    plsc.subcore_barrier()

    pltpu.sync_copy(acc.at[pl.ds(base, RPT)],
                    out_hbm.at[c, pl.ds(base, RPT)])


@functools.cache
def _sc_msg_kernel():
    mesh = plsc.VectorSubcoreMesh(core_axis_name="c", subcore_axis_name="s")
    return pl.kernel(
        _sc_msg_body,
        out_type=jax.ShapeDtypeStruct((NC, NPAD, D), jnp.float32),
        mesh=mesh,
        scratch_types=[
            pltpu.VMEM((NBUF, 2, CHUNK), jnp.int32),
            pltpu.VMEM((NBUF, CHUNK, D), jnp.float32),
            pltpu.VMEM_SHARED((NPAD, D), jnp.float32),
            pltpu.SemaphoreType.DMA,
            pltpu.SemaphoreType.DMA,
            pltpu.SemaphoreType.DMA,
            pltpu.SemaphoreType.DMA,
        ],
    )


def _sc_msg(e, h, idx4):
    return _sc_msg_kernel()(e, h, idx4)


def kernel(x, edge_index, edge_attr, batch, nt_W1, nt_b1, nt_W2, nt_b2,
           et_W1, et_b1, et_W2, et_b2, conv_lin_W, conv_lin_b, conv_eps,
           conv_W1, conv_b1, conv_W2, conv_b2, h_W1, h_b1, h_W2, h_b2):
    f32 = jnp.float32
    src2 = jnp.pad(edge_index[0], (0, EP - E)).reshape(NW, 1, CPW, CHUNK)
    dst2 = jnp.pad(edge_index[1], (0, EP - E),
                   constant_values=N).reshape(NW, 1, CPW, CHUNK)
    idx4 = jnp.concatenate([src2, dst2], axis=1)
    attr = jnp.pad(edge_attr, ((0, EP - E), (0, 0)))

    h0 = pl.pallas_call(
        _nt_body,
        out_shape=jax.ShapeDtypeStruct((N, D), f32),
    )(x, nt_W1, nt_b1.reshape(1, D), nt_W2, nt_b2.reshape(1, D))

    full2 = lambda i: (0, 0)
    full3 = lambda i: (0, 0, 0)
    e_all = pl.pallas_call(
        _et_body,
        grid=(EP // BE,),
        in_specs=[
            pl.BlockSpec((BE, DE), lambda i: (i, 0)),
            pl.BlockSpec((DE, D), full2),
            pl.BlockSpec((1, D), full2),
            pl.BlockSpec((D, D), full2),
            pl.BlockSpec((1, D), full2),
            pl.BlockSpec((3, D, D), full3),
            pl.BlockSpec((3, 1, D), full3),
        ],
        out_specs=[pl.BlockSpec((BE, D), lambda i: (i, 0))] * 3,
        out_shape=[jax.ShapeDtypeStruct((EP, D), f32)] * 3,
    )(attr, et_W1, et_b1.reshape(1, D), et_W2, et_b2.reshape(1, D),
      conv_lin_W, conv_lin_b.reshape(3, 1, D))

    h = h0
    for l in range(2):
        agg = _sc_msg(e_all[l], h, idx4)
        h = pl.pallas_call(
            functools.partial(_upd_body, final_relu=l < 2),
            out_shape=jax.ShapeDtypeStruct((N, D), f32),
        )(h, agg, conv_W1[l], conv_b1[l].reshape(1, D), conv_W2[l],
          conv_b2[l].reshape(1, D),
          (1.0 + conv_eps[l]).reshape(1, 1))

    agg = _sc_msg(e_all[2], h, idx4)
    out = pl.pallas_call(
        _final_body,
        out_shape=jax.ShapeDtypeStruct((G, 1), f32),
    )(h, agg, conv_W1[2], conv_b1[2].reshape(1, D), conv_W2[2],
      conv_b2[2].reshape(1, D), (1.0 + conv_eps[2]).reshape(1, 1),
      batch.reshape(1, N), h_W1, h_b1.reshape(1, D), h_W2, h_b2.reshape(1, 1))
    return out
